# baseline (speedup 1.0000x reference)
import functools
import jax
import jax.numpy as jnp
from jax import lax
from jax.experimental import pallas as pl
from jax.experimental.pallas import tpu as pltpu
from jax.experimental.pallas import tpu_sc as plsc

_HH = 16
_HS = 160
_NSPAN = 4


def _conv_abs_body(x_ref, w_ref, b_ref, out_ref):
    nc = x_ref.shape[1]

    def cbody(c, acc):
        return acc + x_ref[:, c] * w_ref[0, c]

    z = jnp.zeros(
        (x_ref.shape[0], x_ref.shape[2], x_ref.shape[3]), jnp.float32)
    acc = lax.fori_loop(0, nc, cbody, z)
    out_ref[...] = jnp.abs(acc + b_ref[0, 0])


def _select2_body(k, nbits, a_ref, b_ref, out_ref):
    sa = a_ref[...]
    sb = b_ref[...]
    ua = lax.bitcast_convert_type(sa, jnp.int32)
    ub = lax.bitcast_convert_type(sb, jnp.int32)
    bsz = sa.shape[0]

    def body(i, t):
        bit = nbits - 1 - i
        cand = t | jnp.left_shift(jnp.int32(1), bit)
        c = (jnp.sum((ua >= cand[:, :, None]).astype(jnp.int32), axis=(1, 2))
             + jnp.sum((ub >= cand).astype(jnp.int32), axis=1))[:, None]
        return jnp.where(c >= k, cand, t)

    t = lax.fori_loop(0, nbits, body, jnp.zeros((bsz, 1), jnp.int32))
    tf = lax.bitcast_convert_type(t, jnp.float32)
    gta = ua > t[:, :, None]
    gtb = ub > t
    cnt = (jnp.sum(gta.astype(jnp.float32), axis=(1, 2))
           + jnp.sum(gtb.astype(jnp.float32), axis=1))[:, None]
    ssum = (jnp.sum(jnp.where(gta, sa, 0.0), axis=(1, 2))
            + jnp.sum(jnp.where(gtb, sb, 0.0), axis=1))[:, None]
    out_ref[...] = (ssum + (jnp.float32(k) - cnt) * tf) * jnp.float32(1.0 / k)


def _sc_conv_body(h_tc, hspan, wdim, x_ref, w_ref, b_ref, out_ref,
                  xb0, xb1, obuf, wv, bv, sem0, sem1, osem):
    wid = lax.axis_index("s") * 2 + lax.axis_index("c")
    batch = wid // _NSPAN
    span = wid % _NSPAN
    h0 = h_tc + span * hspan
    out0 = span * hspan * wdim

    pltpu.sync_copy(w_ref, wv)
    pltpu.sync_copy(b_ref, bv)

    xbufs = (xb0, xb1)
    sems = (sem0, sem1)

    def start(i):
        return pltpu.async_copy(
            x_ref.at[batch, :, h0 + i, :], xbufs[i % 2], sems[i % 2])

    def compute(xbuf):
        def gbody(g, _):
            col = g * 64

            def cbody(c, accs):
                a0, a1, a2, a3 = accs
                ws = wv[c, :]
                r0 = xbuf[c, pl.ds(col, 16)]
                r1 = xbuf[c, pl.ds(col + 16, 16)]
                r2 = xbuf[c, pl.ds(col + 32, 16)]
                r3 = xbuf[c, pl.ds(col + 48, 16)]
                return (a0 + ws * r0, a1 + ws * r1,
                        a2 + ws * r2, a3 + ws * r3)

            z = jnp.zeros((16,), jnp.float32)
            a0, a1, a2, a3 = lax.fori_loop(0, 96, cbody, (z, z, z, z))
            bvec = bv[...]
            obuf[pl.ds(col, 16)] = jnp.abs(a0 + bvec)
            obuf[pl.ds(col + 16, 16)] = jnp.abs(a1 + bvec)
            obuf[pl.ds(col + 32, 16)] = jnp.abs(a2 + bvec)
            obuf[pl.ds(col + 48, 16)] = jnp.abs(a3 + bvec)
            return 0

        lax.fori_loop(0, wdim // 64, gbody, 0)

    cps = [None, None]
    cps[0] = start(0)
    ocp = None
    for i in range(hspan):
        if i + 1 < hspan:
            cps[(i + 1) % 2] = start(i + 1)
        cps[i % 2].wait()
        if ocp is not None:
            ocp.wait()
        compute(xbufs[i % 2])
        ocp = pltpu.async_copy(
            obuf, out_ref.at[batch, pl.ds(out0 + i * wdim, wdim)], osem)
    ocp.wait()


def kernel(x, W, b):
    B, C, H, Wd = x.shape
    N = H * Wd
    h_tc = H - _HS
    s_sc = _HS * Wd
    hspan = _HS // _NSPAN

    wsm = W.reshape(1, C)
    bb = b.reshape(1, 1)
    wv16 = jnp.broadcast_to(W.reshape(C, 1), (C, 16))
    bv = jnp.broadcast_to(b, (16,))

    s_tc = pl.pallas_call(
        _conv_abs_body,
        grid=(h_tc // _HH,),
        in_specs=[
            pl.BlockSpec((B, C, _HH, Wd), lambda j: (0, 0, j, 0)),
            pl.BlockSpec(memory_space=pltpu.SMEM),
            pl.BlockSpec(memory_space=pltpu.SMEM),
        ],
        out_specs=pl.BlockSpec((B, _HH, Wd), lambda j: (0, j, 0)),
        out_shape=jax.ShapeDtypeStruct((B, h_tc, Wd), jnp.float32),
    )(x, wsm, bb)

    sc_conv = functools.partial(
        pl.kernel,
        out_type=jax.ShapeDtypeStruct((B, s_sc), jnp.float32),
        mesh=plsc.VectorSubcoreMesh(core_axis_name="c", subcore_axis_name="s"),
        scratch_types=[
            pltpu.VMEM((C, Wd), jnp.float32),
            pltpu.VMEM((C, Wd), jnp.float32),
            pltpu.VMEM((Wd,), jnp.float32),
            pltpu.VMEM((C, 16), jnp.float32),
            pltpu.VMEM((16,), jnp.float32),
            pltpu.SemaphoreType.DMA,
            pltpu.SemaphoreType.DMA,
            pltpu.SemaphoreType.DMA,
        ],
    )(functools.partial(_sc_conv_body, h_tc, hspan, Wd))
    s_sc_arr = sc_conv(x, wv16, bv)

    k = max(int(N * 0.1), 1)
    out = pl.pallas_call(
        functools.partial(_select2_body, k, 31),
        out_shape=jax.ShapeDtypeStruct((B, 1), jnp.float32),
    )(s_tc, s_sc_arr)
    return out


# speedup vs baseline: 1.0382x; 1.0382x over previous
import functools
import jax
import jax.numpy as jnp
from jax import lax
from jax.experimental import pallas as pl
from jax.experimental.pallas import tpu as pltpu
from jax.experimental.pallas import tpu_sc as plsc

_HH = 16
_HS = 96
_NSPAN = 4


def _conv_abs_body(x_ref, w_ref, b_ref, out_ref):
    nc = x_ref.shape[1]

    def cbody(c, acc):
        return acc + x_ref[:, c] * w_ref[0, c]

    z = jnp.zeros(
        (x_ref.shape[0], x_ref.shape[2], x_ref.shape[3]), jnp.float32)
    acc = lax.fori_loop(0, nc, cbody, z)
    out_ref[...] = jnp.abs(acc + b_ref[0, 0])


def _select2_body(k, nbits, a_ref, b_ref, out_ref):
    sa = a_ref[...]
    sb = b_ref[...]
    ua = lax.bitcast_convert_type(sa, jnp.int32)
    ub = lax.bitcast_convert_type(sb, jnp.int32)
    bsz = sa.shape[0]

    def body(i, t):
        bit = nbits - 1 - i
        cand = t | jnp.left_shift(jnp.int32(1), bit)
        c = (jnp.sum((ua >= cand[:, :, None]).astype(jnp.int32), axis=(1, 2))
             + jnp.sum((ub >= cand).astype(jnp.int32), axis=1))[:, None]
        return jnp.where(c >= k, cand, t)

    t = lax.fori_loop(0, nbits, body, jnp.zeros((bsz, 1), jnp.int32))
    tf = lax.bitcast_convert_type(t, jnp.float32)
    gta = ua > t[:, :, None]
    gtb = ub > t
    cnt = (jnp.sum(gta.astype(jnp.float32), axis=(1, 2))
           + jnp.sum(gtb.astype(jnp.float32), axis=1))[:, None]
    ssum = (jnp.sum(jnp.where(gta, sa, 0.0), axis=(1, 2))
            + jnp.sum(jnp.where(gtb, sb, 0.0), axis=1))[:, None]
    out_ref[...] = (ssum + (jnp.float32(k) - cnt) * tf) * jnp.float32(1.0 / k)


def _sc_conv_body(h_tc, hspan, wdim, x_ref, w_ref, b_ref, out_ref,
                  xb0, xb1, obuf, wv, bv, sem0, sem1, osem):
    wid = lax.axis_index("s") * 2 + lax.axis_index("c")
    batch = wid // _NSPAN
    span = wid % _NSPAN
    h0 = h_tc + span * hspan
    out0 = span * hspan * wdim

    pltpu.sync_copy(w_ref, wv)
    pltpu.sync_copy(b_ref, bv)

    xbufs = (xb0, xb1)
    sems = (sem0, sem1)

    def start(i):
        return pltpu.async_copy(
            x_ref.at[batch, :, h0 + i, :], xbufs[i % 2], sems[i % 2])

    def compute(xbuf):
        def gbody(g, _):
            col = g * 64

            def cbody(c, accs):
                a0, a1, a2, a3 = accs
                ws = wv[c, :]
                r0 = xbuf[c, pl.ds(col, 16)]
                r1 = xbuf[c, pl.ds(col + 16, 16)]
                r2 = xbuf[c, pl.ds(col + 32, 16)]
                r3 = xbuf[c, pl.ds(col + 48, 16)]
                return (a0 + ws * r0, a1 + ws * r1,
                        a2 + ws * r2, a3 + ws * r3)

            z = jnp.zeros((16,), jnp.float32)
            a0, a1, a2, a3 = lax.fori_loop(0, 96, cbody, (z, z, z, z))
            bvec = bv[...]
            obuf[pl.ds(col, 16)] = jnp.abs(a0 + bvec)
            obuf[pl.ds(col + 16, 16)] = jnp.abs(a1 + bvec)
            obuf[pl.ds(col + 32, 16)] = jnp.abs(a2 + bvec)
            obuf[pl.ds(col + 48, 16)] = jnp.abs(a3 + bvec)
            return 0

        lax.fori_loop(0, wdim // 64, gbody, 0)

    cps = [None, None]
    cps[0] = start(0)
    ocp = None
    for i in range(hspan):
        if i + 1 < hspan:
            cps[(i + 1) % 2] = start(i + 1)
        cps[i % 2].wait()
        if ocp is not None:
            ocp.wait()
        compute(xbufs[i % 2])
        ocp = pltpu.async_copy(
            obuf, out_ref.at[batch, pl.ds(out0 + i * wdim, wdim)], osem)
    ocp.wait()


def kernel(x, W, b):
    B, C, H, Wd = x.shape
    N = H * Wd
    h_tc = H - _HS
    s_sc = _HS * Wd
    hspan = _HS // _NSPAN

    wsm = W.reshape(1, C)
    bb = b.reshape(1, 1)
    wv16 = jnp.broadcast_to(W.reshape(C, 1), (C, 16))
    bv = jnp.broadcast_to(b, (16,))

    s_tc = pl.pallas_call(
        _conv_abs_body,
        grid=(h_tc // _HH,),
        in_specs=[
            pl.BlockSpec((B, C, _HH, Wd), lambda j: (0, 0, j, 0)),
            pl.BlockSpec(memory_space=pltpu.SMEM),
            pl.BlockSpec(memory_space=pltpu.SMEM),
        ],
        out_specs=pl.BlockSpec((B, _HH, Wd), lambda j: (0, j, 0)),
        out_shape=jax.ShapeDtypeStruct((B, h_tc, Wd), jnp.float32),
    )(x, wsm, bb)

    sc_conv = functools.partial(
        pl.kernel,
        out_type=jax.ShapeDtypeStruct((B, s_sc), jnp.float32),
        mesh=plsc.VectorSubcoreMesh(core_axis_name="c", subcore_axis_name="s"),
        scratch_types=[
            pltpu.VMEM((C, Wd), jnp.float32),
            pltpu.VMEM((C, Wd), jnp.float32),
            pltpu.VMEM((Wd,), jnp.float32),
            pltpu.VMEM((C, 16), jnp.float32),
            pltpu.VMEM((16,), jnp.float32),
            pltpu.SemaphoreType.DMA,
            pltpu.SemaphoreType.DMA,
            pltpu.SemaphoreType.DMA,
        ],
    )(functools.partial(_sc_conv_body, h_tc, hspan, Wd))
    s_sc_arr = sc_conv(x, wv16, bv)

    k = max(int(N * 0.1), 1)
    out = pl.pallas_call(
        functools.partial(_select2_body, k, 31),
        out_shape=jax.ShapeDtypeStruct((B, 1), jnp.float32),
    )(s_tc, s_sc_arr)
    return out
